# Initial kernel scaffold; baseline (speedup 1.0000x reference)
import jax
import jax.numpy as jnp
from jax.experimental import pallas as pl


def kernel(userID, review_date_in_unix, user_table, ts_table, boundaries, ts_mean, ts_std):
    raise NotImplementedError("write your pallas kernel here")


# speedup vs baseline: 2.0022x; 2.0022x over previous
import jax
import jax.numpy as jnp
from jax import lax
from jax.experimental import pallas as pl
from jax.experimental.pallas import tpu as pltpu
from jax.experimental.pallas import tpu_sc as plsc

BATCH = 16384
DIM = 32
NBOUND = 1000
NBPAD = 1024
NC = 2
NS = 16
NW = NC * NS
BPW = BATCH // NW
NCHUNK = 4
CHUNK = BPW // NCHUNK
L = 16


def _body(uid_hbm, ts_hbm, utab_hbm, ttab_hbm, bnd_hbm, mean_hbm, istd_hbm,
          out_hbm,
          idx_v, bkt_v, ts_v, bnds_v, mean_v, istd_v, norm_v,
          urows_v, trows_v, usem, tsem):
    c = lax.axis_index("c")
    s = lax.axis_index("s")
    wid = s * NC + c
    base = wid * BPW

    pltpu.sync_copy(uid_hbm.at[wid], idx_v)
    ucopies = [
        pltpu.async_copy(utab_hbm.at[idx_v.at[k]],
                         urows_v.at[pl.ds(k * CHUNK, CHUNK)], usem)
        for k in range(NCHUNK)
    ]

    pltpu.sync_copy(ts_hbm.at[wid], ts_v)
    pltpu.sync_copy(bnd_hbm, bnds_v)
    pltpu.sync_copy(mean_hbm, mean_v)
    pltpu.sync_copy(istd_hbm, istd_v)
    mean = mean_v[...]
    istd = istd_v[...]

    for p in range(BPW // L):
        k, j = divmod(p, CHUNK // L)
        tf = ts_v[pl.ds(p * L, L)].astype(jnp.float32)
        pos = jnp.zeros((L,), jnp.int32)
        for w in (512, 256, 128, 64, 32, 16, 8, 4, 2, 1):
            probe = plsc.load_gather(bnds_v, [pos + (w - 1)])
            pos = jnp.where(probe <= tf, pos + w, pos)
        bkt_v[k, pl.ds(j * L, L)] = pos
        rows = p * L + jnp.arange(L, dtype=jnp.int32)
        zero = jnp.zeros((L,), jnp.int32)
        plsc.store_scatter(norm_v, [rows, zero], (tf - mean) * istd)

    tcopies = [
        pltpu.async_copy(ttab_hbm.at[bkt_v.at[k]],
                         trows_v.at[pl.ds(k * CHUNK, CHUNK)], tsem)
        for k in range(NCHUNK)
    ]

    for cpy in ucopies:
        cpy.wait()
    pltpu.sync_copy(urows_v, out_hbm.at[pl.ds(base, BPW), pl.ds(0, DIM)])
    for cpy in tcopies:
        cpy.wait()
    pltpu.sync_copy(trows_v, out_hbm.at[pl.ds(base, BPW), pl.ds(DIM, DIM)])
    pltpu.sync_copy(norm_v, out_hbm.at[pl.ds(base, BPW), pl.ds(2 * DIM, 1)])


def kernel(userID, review_date_in_unix, user_table, ts_table, boundaries,
           ts_mean, ts_std):
    uid = userID.reshape(NW, NCHUNK, CHUNK)
    ts = review_date_in_unix.reshape(NW, BPW)
    bpad = jnp.concatenate([
        boundaries.astype(jnp.float32),
        jnp.full((NBPAD - NBOUND,), jnp.inf, jnp.float32),
    ])
    mean_v = jnp.broadcast_to(ts_mean.astype(jnp.float32), (L,))
    istd_v = jnp.broadcast_to((1.0 / ts_std).astype(jnp.float32), (L,))

    mesh = plsc.VectorSubcoreMesh(core_axis_name="c", subcore_axis_name="s")
    run = pl.kernel(
        _body,
        out_type=jax.ShapeDtypeStruct((BATCH, 2 * DIM + 1), jnp.float32),
        mesh=mesh,
        scratch_types=[
            pltpu.VMEM((NCHUNK, CHUNK), jnp.int32),
            pltpu.VMEM((NCHUNK, CHUNK), jnp.int32),
            pltpu.VMEM((BPW,), jnp.int32),
            pltpu.VMEM((NBPAD,), jnp.float32),
            pltpu.VMEM((L,), jnp.float32),
            pltpu.VMEM((L,), jnp.float32),
            pltpu.VMEM((BPW, 1), jnp.float32),
            pltpu.VMEM((BPW, DIM), jnp.float32),
            pltpu.VMEM((BPW, DIM), jnp.float32),
            pltpu.SemaphoreType.DMA,
            pltpu.SemaphoreType.DMA,
        ],
        compiler_params=pltpu.CompilerParams(use_tc_tiling_on_sc=False,
                                             needs_layout_passes=False),
    )
    return run(uid, ts, user_table, ts_table, bpad, mean_v, istd_v)


# speedup vs baseline: 2.4438x; 1.2206x over previous
import jax
import jax.numpy as jnp
from jax import lax
from jax.experimental import pallas as pl
from jax.experimental.pallas import tpu as pltpu
from jax.experimental.pallas import tpu_sc as plsc

BATCH = 16384
DIM = 32
NBOUND = 1000
NBPAD = 1024
NC = 2
NS = 16
NW = NC * NS
BPW = BATCH // NW
L = 16
OUTW = 128
NBUF = 16
TSROWS = NBOUND + 8


def _body(uid_hbm, ts_hbm, utab_hbm, ttab_hbm, bnd_hbm, mean_hbm, istd_hbm,
          out_hbm,
          idx_v, bkt_v, ts_v, bnds_v, mean_v, istd_v,
          out_v, ring, *sems):
    c = lax.axis_index("c")
    s = lax.axis_index("s")
    wid = s * NC + c
    base = wid * BPW

    pltpu.sync_copy(uid_hbm.at[wid], idx_v)

    def make_phase(tab_hbm, idx_ref, col0):
        def idx_vec(i0):
            return idx_ref[0, pl.ds(i0, L)]

        def fire(row, b):
            off = pl.multiple_of((row >> 3) * 8, 8)
            pltpu.async_copy(tab_hbm.at[pl.ds(off, 8)], ring.at[b], sems[b])

        def drain(b):
            pltpu.make_async_copy(tab_hbm.at[pl.ds(0, 8)], ring.at[b],
                                  sems[b]).wait()

        def consume(i, sub, b):
            out_v[i, pl.ds(col0, L)] = ring[b, sub, pl.ds(0, L)]
            out_v[i, pl.ds(col0 + L, L)] = ring[b, sub, pl.ds(L, L)]

        return idx_vec, fire, drain, consume

    uvec, ufire, udrain, uconsume = make_phase(utab_hbm, idx_v, 0)
    tvec, tfire, tdrain, tconsume = make_phase(ttab_hbm, bkt_v, DIM)

    vec0 = uvec(0)
    for b in range(NBUF):
        ufire(vec0[b], b)

    pltpu.sync_copy(ts_hbm.at[wid], ts_v)
    pltpu.sync_copy(bnd_hbm, bnds_v)
    pltpu.sync_copy(mean_hbm, mean_v)
    pltpu.sync_copy(istd_hbm, istd_v)
    mean = mean_v[...]
    istd = istd_v[...]

    norm_col = jnp.full((L,), 2 * DIM, jnp.int32)
    for p in range(BPW // L):
        tf = ts_v[0, pl.ds(p * L, L)].astype(jnp.float32)
        pos = jnp.zeros((L,), jnp.int32)
        for w in (512, 256, 128, 64, 32, 16, 8, 4, 2, 1):
            probe = plsc.load_gather(bnds_v, [pos + (w - 1)])
            pos = jnp.where(probe <= tf, pos + w, pos)
        bkt_v[0, pl.ds(p * L, L)] = pos
        rows = p * L + jnp.arange(L, dtype=jnp.int32)
        plsc.store_scatter(out_v, [rows, norm_col], (tf - mean) * istd)

    nblk = BPW // NBUF

    def ubody(g, carry):
        i0 = g * NBUF
        cur = uvec(i0)
        nxt = uvec(i0 + NBUF)
        for b in range(NBUF):
            udrain(b)
            uconsume(i0 + b, cur[b] & 7, b)
            ufire(nxt[b], b)
        return carry
    lax.fori_loop(0, nblk - 1, ubody, 0)
    lastu = uvec((nblk - 1) * NBUF)
    tvec0 = tvec(0)
    for b in range(NBUF):
        udrain(b)
        uconsume((nblk - 1) * NBUF + b, lastu[b] & 7, b)
        tfire(tvec0[b], b)

    def tbody(g, carry):
        i0 = g * NBUF
        cur = tvec(i0)
        nxt = tvec(i0 + NBUF)
        for b in range(NBUF):
            tdrain(b)
            tconsume(i0 + b, cur[b] & 7, b)
            tfire(nxt[b], b)
        return carry
    lax.fori_loop(0, nblk - 1, tbody, 0)
    lastt = tvec((nblk - 1) * NBUF)
    for b in range(NBUF):
        tdrain(b)
        tconsume((nblk - 1) * NBUF + b, lastt[b] & 7, b)

    pltpu.sync_copy(out_v, out_hbm.at[pl.ds(base, BPW)])


def kernel(userID, review_date_in_unix, user_table, ts_table, boundaries,
           ts_mean, ts_std):
    uid = userID.reshape(NW, 1, BPW)
    ts = review_date_in_unix.reshape(NW, 1, BPW)
    ttab = jnp.pad(ts_table, ((0, TSROWS - (NBOUND + 1)), (0, 0)))
    bpad = jnp.concatenate([
        boundaries.astype(jnp.float32),
        jnp.full((NBPAD - NBOUND,), jnp.inf, jnp.float32),
    ])
    mean_v = jnp.broadcast_to(ts_mean.astype(jnp.float32), (L,))
    istd_v = jnp.broadcast_to((1.0 / ts_std).astype(jnp.float32), (L,))

    mesh = plsc.VectorSubcoreMesh(core_axis_name="c", subcore_axis_name="s")
    run = pl.kernel(
        _body,
        out_type=jax.ShapeDtypeStruct((BATCH, OUTW), jnp.float32),
        mesh=mesh,
        scratch_types=[
            pltpu.VMEM((1, BPW), jnp.int32),
            pltpu.VMEM((1, BPW), jnp.int32),
            pltpu.VMEM((1, BPW), jnp.int32),
            pltpu.VMEM((NBPAD,), jnp.float32),
            pltpu.VMEM((L,), jnp.float32),
            pltpu.VMEM((L,), jnp.float32),
            pltpu.VMEM((BPW, OUTW), jnp.float32),
            pltpu.VMEM((NBUF, 8, DIM), jnp.float32),
        ] + [pltpu.SemaphoreType.DMA] * NBUF,
        compiler_params=pltpu.CompilerParams(needs_layout_passes=False),
    )
    out = run(uid, ts, user_table, ttab, bpad, mean_v, istd_v)
    return out[:, : 2 * DIM + 1]
